# Initial kernel scaffold; baseline (speedup 1.0000x reference)
import jax
import jax.numpy as jnp
from jax.experimental import pallas as pl


def kernel(atom_num, dis1, dis2, id1u, id1v, id2u, id2v, params):
    raise NotImplementedError("write your pallas kernel here")


# speedup vs baseline: 2.1112x; 2.1112x over previous
import functools

import jax
import jax.numpy as jnp
from jax import lax
from jax.experimental import pallas as pl
from jax.experimental.pallas import tpu as pltpu
from jax.experimental.pallas import tpu_sc as plsc

HD = 128
N = 10000
E = 320000

NC = 2
NS = 16
NW = NC * NS
CHUNK = 128
NBLK_E = E // CHUNK
NPAD = 10240
NBLK_A = NPAD // CHUNK

@functools.cache
def _sc_mesh():
    return plsc.VectorSubcoreMesh(core_axis_name="c", subcore_axis_name="s",
                                  num_cores=NC, num_subcores=NS)


def _wid():
    return lax.axis_index("s") * NC + lax.axis_index("c")


def _strided_blocks(wid, nblk, body):
    nloc = (nblk - wid + NW - 1) // NW

    def loop_body(k, _):
        body(wid + k * NW)
        return 0

    lax.fori_loop(0, nloc, loop_body, 0)


@functools.cache
def _emb_gather_call():
    @functools.partial(
        pl.kernel,
        out_type=jax.ShapeDtypeStruct((NPAD, HD), jnp.float32),
        mesh=_sc_mesh(),
        scratch_types=[
            pltpu.VMEM((CHUNK,), jnp.int32),
            pltpu.VMEM((CHUNK, HD), jnp.float32),
            pltpu.SemaphoreType.DMA,
        ],
    )
    def _emb_gather(tab_hbm, idx_hbm, out_hbm, idx_v, rows_v, sem):
        wid = _wid()

        def body(b):
            base = b * CHUNK
            pltpu.sync_copy(idx_hbm.at[pl.ds(base, CHUNK)], idx_v)
            pltpu.async_copy(tab_hbm.at[idx_v], rows_v, sem).wait()
            pltpu.sync_copy(rows_v, out_hbm.at[pl.ds(base, CHUNK)])

        _strided_blocks(wid, NBLK_A, body)

    return _emb_gather


@functools.cache
def _edge_gather_call():
    @functools.partial(
        pl.kernel,
        out_type=[jax.ShapeDtypeStruct((E, HD), jnp.float32),
                  jax.ShapeDtypeStruct((E, HD), jnp.float32)],
        mesh=_sc_mesh(),
        scratch_types=[
            pltpu.VMEM((CHUNK,), jnp.int32),
            pltpu.VMEM((CHUNK, HD), jnp.float32),
            pltpu.SemaphoreType.DMA,
        ],
    )
    def _edge_gather(tab_u, tab_v, idx_u, idx_v, out_u, out_v,
                     idx_b, rows_b, sem):
        wid = _wid()

        def body(b):
            base = b * CHUNK
            pltpu.sync_copy(idx_u.at[pl.ds(base, CHUNK)], idx_b)
            pltpu.async_copy(tab_u.at[idx_b], rows_b, sem).wait()
            pltpu.sync_copy(rows_b, out_u.at[pl.ds(base, CHUNK)])
            pltpu.sync_copy(idx_v.at[pl.ds(base, CHUNK)], idx_b)
            pltpu.async_copy(tab_v.at[idx_b], rows_b, sem).wait()
            pltpu.sync_copy(rows_b, out_v.at[pl.ds(base, CHUNK)])

        _strided_blocks(wid, NBLK_E, body)

    return _edge_gather


_ZROWS = NPAD // NS


@functools.cache
def _edge_scatter_call():
    @functools.partial(
        pl.kernel,
        out_type=jax.ShapeDtypeStruct((NC, NPAD, HD), jnp.float32),
        mesh=_sc_mesh(),
        scratch_types=[
            pltpu.VMEM((CHUNK,), jnp.int32),
            pltpu.VMEM((CHUNK, HD), jnp.float32),
            pltpu.VMEM_SHARED((NPAD, HD), jnp.float32),
            pltpu.SemaphoreType.DMA,
        ],
    )
    def _edge_scatter(m_hbm, idx_hbm, zeros_hbm, out_hbm,
                      idx_b, rows_b, acc, sem):
        cid = lax.axis_index("c")
        sid = lax.axis_index("s")
        wid = sid * NC + cid

        for j in range(_ZROWS // CHUNK):
            o = sid * _ZROWS + j * CHUNK
            pltpu.sync_copy(zeros_hbm.at[pl.ds(o, CHUNK)], rows_b)
            pltpu.sync_copy(rows_b, acc.at[pl.ds(o, CHUNK)])
        plsc.subcore_barrier()

        def body(b):
            base = b * CHUNK
            pltpu.sync_copy(idx_hbm.at[pl.ds(base, CHUNK)], idx_b)
            pltpu.sync_copy(m_hbm.at[pl.ds(base, CHUNK)], rows_b)
            pltpu.sync_copy(rows_b, acc.at[idx_b], add=True)

        _strided_blocks(wid, NBLK_E, body)
        plsc.subcore_barrier()

        for j in range(_ZROWS // CHUNK):
            o = sid * _ZROWS + j * CHUNK
            pltpu.sync_copy(acc.at[pl.ds(o, CHUNK)], rows_b)
            pltpu.sync_copy(rows_b, out_hbm.at[cid, pl.ds(o, CHUNK)])

    return _edge_scatter


def _silu(x):
    return x * jax.nn.sigmoid(x)


def _mlp_body(ru, rv, dis, w1c, b1, W2, b2, W3, b3, out):
    x = ru[...] + rv[...] + dis[...] * w1c[...] + b1[...]
    x = _silu(x)
    x = jnp.dot(x, W2[...], preferred_element_type=jnp.float32) + b2[...]
    x = _silu(x)
    out[...] = jnp.dot(x, W3[...], preferred_element_type=jnp.float32) + b3[...]


_BE = 512


def _edge_mlp(ru, rv, dis, w1c, b1, W2, b2, W3, b3):
    full = lambda i: (0, 0)
    return pl.pallas_call(
        _mlp_body,
        grid=(E // _BE,),
        in_specs=[
            pl.BlockSpec((_BE, HD), lambda i: (i, 0)),
            pl.BlockSpec((_BE, HD), lambda i: (i, 0)),
            pl.BlockSpec((_BE, 1), lambda i: (i, 0)),
            pl.BlockSpec((1, HD), full),
            pl.BlockSpec((1, HD), full),
            pl.BlockSpec((HD, HD), full),
            pl.BlockSpec((1, HD), full),
            pl.BlockSpec((HD, HD), full),
            pl.BlockSpec((1, HD), full),
        ],
        out_specs=pl.BlockSpec((_BE, HD), lambda i: (i, 0)),
        out_shape=jax.ShapeDtypeStruct((E, HD), jnp.float32),
    )(ru, rv, dis, w1c, b1, W2, b2, W3, b3)


_BN = 2048


def _upd_body(nout, h, a1, a2, W1h, W1a, W1b, b1, W2, b2, *rest):
    nexts = rest[:2 * nout]
    outs = rest[2 * nout:]
    href = h[...]
    x = (jnp.dot(href, W1h[...], preferred_element_type=jnp.float32)
         + jnp.dot(a1[0] + a1[1], W1a[...], preferred_element_type=jnp.float32)
         + jnp.dot(a2[0] + a2[1], W1b[...], preferred_element_type=jnp.float32)
         + b1[...])
    x = _silu(x)
    hn = href + jnp.dot(x, W2[...], preferred_element_type=jnp.float32) + b2[...]
    outs[0][...] = hn
    for k in range(nout):
        W, b = nexts[2 * k], nexts[2 * k + 1]
        outs[k + 1][...] = (jnp.dot(hn, W[...], preferred_element_type=jnp.float32)
                            + b[...])


def _atom_update(h, a1, a2, p, next_mats):
    full = lambda i: (0, 0)
    nout = len(next_mats)
    in_specs = [
        pl.BlockSpec((_BN, HD), lambda i: (i, 0)),
        pl.BlockSpec((NC, _BN, HD), lambda i: (0, i, 0)),
        pl.BlockSpec((NC, _BN, HD), lambda i: (0, i, 0)),
        pl.BlockSpec((HD, HD), full),
        pl.BlockSpec((HD, HD), full),
        pl.BlockSpec((HD, HD), full),
        pl.BlockSpec((1, HD), full),
        pl.BlockSpec((HD, HD), full),
        pl.BlockSpec((1, HD), full),
    ]
    args = [h, a1, a2, p['W1'][:HD], p['W1'][HD:2 * HD], p['W1'][2 * HD:],
            p['b1'][None], p['W2'], p['b2'][None]]
    out_shapes = [jax.ShapeDtypeStruct((NPAD, HD), jnp.float32)]
    out_specs = [pl.BlockSpec((_BN, HD), lambda i: (i, 0))]
    for W, b in next_mats:
        K = W.shape[1]
        in_specs += [pl.BlockSpec((HD, K), full), pl.BlockSpec((1, K), full)]
        args += [W, b]
        out_shapes.append(jax.ShapeDtypeStruct((NPAD, K), jnp.float32))
        out_specs.append(pl.BlockSpec((_BN, K), lambda i: (i, 0)))
    return pl.pallas_call(
        functools.partial(_upd_body, nout),
        grid=(NPAD // _BN,),
        in_specs=in_specs,
        out_specs=out_specs,
        out_shape=out_shapes,
    )(*args)


def _pre_body(h, Wa, Wb, Wc, Wd, oa, ob, oc, od):
    href = h[...]
    oa[...] = jnp.dot(href, Wa[...], preferred_element_type=jnp.float32)
    ob[...] = jnp.dot(href, Wb[...], preferred_element_type=jnp.float32)
    oc[...] = jnp.dot(href, Wc[...], preferred_element_type=jnp.float32)
    od[...] = jnp.dot(href, Wd[...], preferred_element_type=jnp.float32)


def _precompute_tables(h, p1, p2):
    full = lambda i: (0, 0)
    return pl.pallas_call(
        _pre_body,
        grid=(NPAD // _BN,),
        in_specs=[pl.BlockSpec((_BN, HD), lambda i: (i, 0))] +
                 [pl.BlockSpec((HD, HD), full)] * 4,
        out_specs=[pl.BlockSpec((_BN, HD), lambda i: (i, 0))] * 4,
        out_shape=[jax.ShapeDtypeStruct((NPAD, HD), jnp.float32)] * 4,
    )(h, p1['W1'][:HD], p1['W1'][HD:2 * HD], p2['W1'][:HD], p2['W1'][HD:2 * HD])


def _round(h, tabs, dis1, dis2, id1u, id1v, id2u, id2v, p1, p2, pupd,
           zeros, next_mats):
    g1u, g1v, g2u, g2v = tabs
    r1u, r1v = _edge_gather_call()(g1u, g1v, id1u, id1v)
    r2u, r2v = _edge_gather_call()(g2u, g2v, id2u, id2v)
    m1 = _edge_mlp(r1u, r1v, dis1, p1['W1'][2 * HD:], p1['b1'][None],
                   p1['W2'], p1['b2'][None], p1['W3'], p1['b3'][None])
    m2 = _edge_mlp(r2u, r2v, dis2, p2['W1'][2 * HD:], p2['b1'][None],
                   p2['W2'], p2['b2'][None], p2['W3'], p2['b3'][None])
    a1 = _edge_scatter_call()(m1, id1v, zeros)
    a2 = _edge_scatter_call()(m2, id2v, zeros)
    return _atom_update(h, a1, a2, pupd, next_mats)


def kernel(atom_num, dis1, dis2, id1u, id1v, id2u, id2v, params):
    p = params
    i32 = jnp.int32
    id1u, id1v = id1u.astype(i32), id1v.astype(i32)
    id2u, id2v = id2u.astype(i32), id2v.astype(i32)
    an = jnp.pad(atom_num.astype(i32), (0, NPAD - N))
    dis1 = dis1[:, None]
    dis2 = dis2[:, None]
    zeros = jnp.zeros((NPAD, HD), jnp.float32)

    h = _emb_gather_call()(p['atom_emb'], an)
    tabs1 = _precompute_tables(h, p['edge1'], p['edge2'])
    h2, g1u, g1v, g2u, g2v = _round(
        h, tabs1, dis1, dis2, id1u, id1v, id2u, id2v,
        p['edge1'], p['edge2'], p['upd1'], zeros,
        [(p['uedge1']['W1'][:HD], jnp.zeros((1, HD), jnp.float32)),
         (p['uedge1']['W1'][HD:2 * HD], jnp.zeros((1, HD), jnp.float32)),
         (p['uedge2']['W1'][:HD], jnp.zeros((1, HD), jnp.float32)),
         (p['uedge2']['W1'][HD:2 * HD], jnp.zeros((1, HD), jnp.float32))])
    (delta,) = _round(
        h2, (g1u, g1v, g2u, g2v), dis1, dis2, id1u, id1v, id2u, id2v,
        p['uedge1'], p['uedge2'], p['upd2'], zeros,
        [(p['Wout'], p['bout'][None])])[1:]
    return delta[:N]
